# Initial kernel scaffold; baseline (speedup 1.0000x reference)
import functools

import jax
import jax.numpy as jnp
from jax import lax
from jax.experimental import pallas as pl
from jax.experimental.pallas import tpu as pltpu
from jax.experimental.pallas import tpu_sc as plsc

VOCAB = 1000
D = 1000
B, T = 4096, 50
N_ROWS = B * T
NC, NS = 2, 16
NW = NC * NS
ROWS_PER_W = N_ROWS // NW
CHUNK = 40
N_CHUNKS = ROWS_PER_W // CHUNK


def _sc_gather(table, idx3):
    mesh = plsc.VectorSubcoreMesh(core_axis_name="c", subcore_axis_name="s")

    @functools.partial(
        pl.kernel,
        mesh=mesh,
        out_type=jax.ShapeDtypeStruct((N_ROWS, D), jnp.float32),
        scratch_types=[
            pltpu.VMEM((N_CHUNKS, CHUNK), jnp.int32),
            pltpu.VMEM((CHUNK, D), jnp.float32),
            pltpu.SemaphoreType.DMA,
        ],
    )
    def k(table_hbm, idx_hbm, out_hbm, idx_v, rows_v, gsem):
        wid = lax.axis_index("s") * NC + lax.axis_index("c")
        base = wid * ROWS_PER_W
        pltpu.sync_copy(idx_hbm.at[wid], idx_v)

        def body(j, carry):
            pltpu.async_copy(table_hbm.at[idx_v.at[j]], rows_v, gsem).wait()
            pltpu.sync_copy(rows_v, out_hbm.at[pl.ds(base + j * CHUNK, CHUNK)])
            return carry

        lax.fori_loop(0, N_CHUNKS, body, 0)

    return k(table, idx3)


def kernel(idx, table):
    flat = idx.reshape(-1).astype(jnp.int32)
    idx3 = flat.reshape(NW, N_CHUNKS, CHUNK)
    out = _sc_gather(table, idx3)
    return out.reshape(B, T, D)


# speedup vs baseline: 1.0018x; 1.0018x over previous
import functools

import jax
import jax.numpy as jnp
from jax import lax
from jax.experimental import pallas as pl
from jax.experimental.pallas import tpu as pltpu
from jax.experimental.pallas import tpu_sc as plsc

VOCAB = 1000
D = 1000
B, T = 4096, 50
N_ROWS = B * T
NC, NS = 2, 16
NW = NC * NS
ROWS_PER_W = N_ROWS // NW
CHUNK = 40
N_CHUNKS = ROWS_PER_W // CHUNK


def _sc_gather(table, idx3):
    mesh = plsc.VectorSubcoreMesh(core_axis_name="c", subcore_axis_name="s")

    @functools.partial(
        pl.kernel,
        mesh=mesh,
        out_type=jax.ShapeDtypeStruct((N_ROWS, D), jnp.float32),
        compiler_params=pltpu.CompilerParams(use_tc_tiling_on_sc=False),
        scratch_types=[
            pltpu.VMEM((N_CHUNKS, CHUNK), jnp.int32),
            pltpu.VMEM((CHUNK, D), jnp.float32),
            pltpu.SemaphoreType.DMA,
        ],
    )
    def k(table_hbm, idx_hbm, out_hbm, idx_v, rows_v, gsem):
        wid = lax.axis_index("s") * NC + lax.axis_index("c")
        base = wid * ROWS_PER_W
        pltpu.sync_copy(idx_hbm.at[wid], idx_v)

        def body(j, carry):
            pltpu.async_copy(table_hbm.at[idx_v.at[j]], rows_v, gsem).wait()
            pltpu.sync_copy(rows_v, out_hbm.at[pl.ds(base + j * CHUNK, CHUNK)])
            return carry

        lax.fori_loop(0, N_CHUNKS, body, 0)

    return k(table, idx3)


def kernel(idx, table):
    flat = idx.reshape(-1).astype(jnp.int32)
    idx3 = flat.reshape(NW, N_CHUNKS, CHUNK)
    out = _sc_gather(table, idx3)
    return out.reshape(B, T, D)


# speedup vs baseline: 1.0339x; 1.0320x over previous
import functools

import jax
import jax.numpy as jnp
from jax import lax
from jax.experimental import pallas as pl
from jax.experimental.pallas import tpu as pltpu
from jax.experimental.pallas import tpu_sc as plsc

VOCAB = 1000
D = 1000
B, T = 4096, 50
N_ROWS = B * T
NC, NS = 2, 16
NW = NC * NS
ROWS_PER_W = N_ROWS // NW
CHUNK = 40
N_CHUNKS = ROWS_PER_W // CHUNK


def _sc_gather(table, idx3):
    mesh = plsc.VectorSubcoreMesh(core_axis_name="c", subcore_axis_name="s")

    @functools.partial(
        pl.kernel,
        mesh=mesh,
        out_type=jax.ShapeDtypeStruct((N_ROWS, D), jnp.float32),
        compiler_params=pltpu.CompilerParams(use_tc_tiling_on_sc=False),
        scratch_types=[
            pltpu.VMEM((N_CHUNKS, CHUNK), jnp.int32),
            pltpu.VMEM((CHUNK, D), jnp.float32),
            pltpu.VMEM((CHUNK, D), jnp.float32),
            pltpu.SemaphoreType.DMA,
            pltpu.SemaphoreType.DMA,
            pltpu.SemaphoreType.DMA,
            pltpu.SemaphoreType.DMA,
        ],
    )
    def k(table_hbm, idx_hbm, out_hbm, idx_v, buf0, buf1, g0, g1, w0, w1):
        wid = lax.axis_index("s") * NC + lax.axis_index("c")
        base = wid * ROWS_PER_W
        pltpu.sync_copy(idx_hbm.at[wid], idx_v)

        bufs = (buf0, buf1)
        gsems = (g0, g1)
        wsems = (w0, w1)

        def start_gather(j, b):
            pltpu.async_copy(table_hbm.at[idx_v.at[j]], bufs[b], gsems[b])

        def wait_gather(j, b):
            pltpu.make_async_copy(
                table_hbm.at[idx_v.at[j]], bufs[b], gsems[b]).wait()

        def start_write(j, b):
            pltpu.async_copy(
                bufs[b], out_hbm.at[pl.ds(base + j * CHUNK, CHUNK)], wsems[b])

        def wait_write(j, b):
            pltpu.make_async_copy(
                bufs[b], out_hbm.at[pl.ds(base + j * CHUNK, CHUNK)],
                wsems[b]).wait()

        start_gather(0, 0)
        wait_gather(0, 0)
        start_write(0, 0)
        start_gather(1, 1)

        def step(j, b):
            wait_gather(j, b)
            start_write(j, b)
            wait_write(j - 1, 1 - b)
            start_gather(j + 1, 1 - b)

        def body(jj, carry):
            step(2 * jj + 1, 1)
            step(2 * jj + 2, 0)
            return carry

        lax.fori_loop(0, (N_CHUNKS - 4) // 2, body, 0)

        step(N_CHUNKS - 3, 1)
        step(N_CHUNKS - 2, 0)
        j_last = N_CHUNKS - 1
        wait_gather(j_last, 1)
        start_write(j_last, 1)
        wait_write(j_last - 1, 0)
        wait_write(j_last, 1)

    return k(table, idx3)


def kernel(idx, table):
    flat = idx.reshape(-1).astype(jnp.int32)
    idx3 = flat.reshape(NW, N_CHUNKS, CHUNK)
    out = _sc_gather(table, idx3)
    return out.reshape(B, T, D)


# speedup vs baseline: 1.1530x; 1.1153x over previous
import functools

import jax
import jax.numpy as jnp
from jax import lax
from jax.experimental import pallas as pl
from jax.experimental.pallas import tpu as pltpu
from jax.experimental.pallas import tpu_sc as plsc

VOCAB = 1000
D = 1000
B, T = 4096, 50
N_ROWS = B * T
NC, NS = 2, 16
NW = NC * NS
ROWS_PER_W = N_ROWS // NW
CHUNK = 16
N_CHUNKS = ROWS_PER_W // CHUNK


def _sc_gather(table, idx3):
    mesh = plsc.VectorSubcoreMesh(core_axis_name="c", subcore_axis_name="s")

    @functools.partial(
        pl.kernel,
        mesh=mesh,
        out_type=jax.ShapeDtypeStruct((N_ROWS, D), jnp.float32),
        compiler_params=pltpu.CompilerParams(use_tc_tiling_on_sc=False),
        scratch_types=[
            pltpu.VMEM((N_CHUNKS, CHUNK), jnp.int32),
            pltpu.VMEM((CHUNK, D), jnp.float32),
            pltpu.VMEM((CHUNK, D), jnp.float32),
            pltpu.VMEM_SHARED((VOCAB, D), jnp.float32),
            pltpu.SemaphoreType.DMA,
            pltpu.SemaphoreType.DMA,
            pltpu.SemaphoreType.DMA,
            pltpu.SemaphoreType.DMA,
        ],
    )
    def k(table_hbm, idx_hbm, out_hbm, idx_v, buf0, buf1, table_sp,
          g0, g1, w0, w1):
        sid = lax.axis_index("s")
        wid = sid * NC + lax.axis_index("c")
        base = wid * ROWS_PER_W
        pltpu.sync_copy(idx_hbm.at[wid], idx_v)

        @pl.when(sid == 0)
        def _():
            pltpu.sync_copy(table_hbm, table_sp)

        plsc.subcore_barrier()

        bufs = (buf0, buf1)
        gsems = (g0, g1)
        wsems = (w0, w1)

        def start_gather(j, b):
            pltpu.async_copy(table_sp.at[idx_v.at[j]], bufs[b], gsems[b])

        def wait_gather(j, b):
            pltpu.make_async_copy(
                table_sp.at[idx_v.at[j]], bufs[b], gsems[b]).wait()

        def start_write(j, b):
            pltpu.async_copy(
                bufs[b], out_hbm.at[pl.ds(base + j * CHUNK, CHUNK)], wsems[b])

        def wait_write(j, b):
            pltpu.make_async_copy(
                bufs[b], out_hbm.at[pl.ds(base + j * CHUNK, CHUNK)],
                wsems[b]).wait()

        start_gather(0, 0)
        wait_gather(0, 0)
        start_write(0, 0)
        start_gather(1, 1)

        def step(j, b):
            wait_gather(j, b)
            start_write(j, b)
            wait_write(j - 1, 1 - b)
            start_gather(j + 1, 1 - b)

        def body(jj, carry):
            step(2 * jj + 1, 1)
            step(2 * jj + 2, 0)
            return carry

        lax.fori_loop(0, (N_CHUNKS - 4) // 2, body, 0)

        step(N_CHUNKS - 3, 1)
        step(N_CHUNKS - 2, 0)
        j_last = N_CHUNKS - 1
        wait_gather(j_last, 1)
        start_write(j_last, 1)
        wait_write(j_last - 1, 0)
        wait_write(j_last, 1)

    return k(table, idx3)


def kernel(idx, table):
    flat = idx.reshape(-1).astype(jnp.int32)
    idx3 = flat.reshape(NW, N_CHUNKS, CHUNK)
    out = _sc_gather(table, idx3)
    return out.reshape(B, T, D)


# speedup vs baseline: 1.1601x; 1.0061x over previous
import functools

import jax
import jax.numpy as jnp
from jax import lax
from jax.experimental import pallas as pl
from jax.experimental.pallas import tpu as pltpu
from jax.experimental.pallas import tpu_sc as plsc

VOCAB = 1000
D = 1000
B, T = 4096, 50
N_ROWS = B * T
NC, NS = 2, 16
NW = NC * NS
ROWS_PER_W = N_ROWS // NW
CHUNK = 32
N_CHUNKS = ROWS_PER_W // CHUNK
N_HALVES = 2
HALF = N_CHUNKS // N_HALVES


def _sc_gather(table, idx4):
    mesh = plsc.VectorSubcoreMesh(core_axis_name="c", subcore_axis_name="s")

    @functools.partial(
        pl.kernel,
        mesh=mesh,
        out_type=jax.ShapeDtypeStruct((N_ROWS, D), jnp.float32),
        compiler_params=pltpu.CompilerParams(use_tc_tiling_on_sc=False),
        scratch_types=[
            pltpu.VMEM((HALF, CHUNK), jnp.int32),
            pltpu.VMEM((CHUNK, D), jnp.float32),
            pltpu.VMEM((CHUNK, D), jnp.float32),
            pltpu.VMEM_SHARED((VOCAB, D), jnp.float32),
            pltpu.SemaphoreType.DMA,
            pltpu.SemaphoreType.DMA,
            pltpu.SemaphoreType.DMA,
            pltpu.SemaphoreType.DMA,
        ],
    )
    def k(table_hbm, idx_hbm, out_hbm, idx_v, buf0, buf1, table_sp,
          g0, g1, w0, w1):
        sid = lax.axis_index("s")
        wid = sid * NC + lax.axis_index("c")
        base = wid * ROWS_PER_W

        @pl.when(sid == 0)
        def _():
            pltpu.sync_copy(table_hbm, table_sp)

        plsc.subcore_barrier()

        bufs = (buf0, buf1)
        gsems = (g0, g1)
        wsems = (w0, w1)

        def half_pass(h):
            pltpu.sync_copy(idx_hbm.at[wid, h], idx_v)
            off0 = base + h * HALF * CHUNK

            def start_gather(j, b):
                pltpu.async_copy(table_sp.at[idx_v.at[j]], bufs[b], gsems[b])

            def wait_gather(j, b):
                pltpu.make_async_copy(
                    table_sp.at[idx_v.at[j]], bufs[b], gsems[b]).wait()

            def start_write(j, b):
                pltpu.async_copy(
                    bufs[b], out_hbm.at[pl.ds(off0 + j * CHUNK, CHUNK)],
                    wsems[b])

            def wait_write(j, b):
                pltpu.make_async_copy(
                    bufs[b], out_hbm.at[pl.ds(off0 + j * CHUNK, CHUNK)],
                    wsems[b]).wait()

            def step(j, b):
                wait_gather(j, b)
                start_write(j, b)
                wait_write(j - 1, 1 - b)
                start_gather(j + 1, 1 - b)

            start_gather(0, 0)
            wait_gather(0, 0)
            start_write(0, 0)
            start_gather(1, 1)

            def body(jj, carry):
                step(2 * jj + 1, 1)
                step(2 * jj + 2, 0)
                return carry

            lax.fori_loop(0, (HALF - 4) // 2, body, 0)

            step(HALF - 3, 1)
            step(HALF - 2, 0)
            j_last = HALF - 1
            wait_gather(j_last, 1)
            start_write(j_last, 1)
            wait_write(j_last - 1, 0)
            wait_write(j_last, 1)

        for h in range(N_HALVES):
            half_pass(h)

    return k(table, idx4)


def kernel(idx, table):
    flat = idx.reshape(-1).astype(jnp.int32)
    idx4 = flat.reshape(NW, N_HALVES, HALF, CHUNK)
    out = _sc_gather(table, idx4)
    return out.reshape(B, T, D)
